# baseline (speedup 1.0000x reference)
import jax
import jax.numpy as jnp
from jax.experimental import pallas as pl
from jax.experimental.pallas import tpu as pltpu


def _dma_body(fm_hbm, lm_hbm, fo_hbm, lo_hbm, sem_f, sem_l):
    batch = fo_hbm.shape[0]
    cf = pltpu.make_async_copy(fm_hbm.at[pl.ds(0, batch), :], fo_hbm, sem_f)
    cl = pltpu.make_async_copy(lm_hbm.at[pl.ds(0, batch), :], lo_hbm, sem_l)
    cf.start()
    cl.start()
    cf.wait()
    cl.wait()


def kernel(features, labels, features_memory, labels_memory):
    batch = features.shape[0]
    dim = features_memory.shape[1]
    feats_out, labels_out = pl.pallas_call(
        _dma_body,
        out_shape=(
            jax.ShapeDtypeStruct((batch, dim), features_memory.dtype),
            jax.ShapeDtypeStruct((batch, 1), labels_memory.dtype),
        ),
        in_specs=[
            pl.BlockSpec(memory_space=pl.ANY),
            pl.BlockSpec(memory_space=pl.ANY),
        ],
        out_specs=(
            pl.BlockSpec(memory_space=pl.ANY),
            pl.BlockSpec(memory_space=pl.ANY),
        ),
        scratch_shapes=[pltpu.SemaphoreType.DMA, pltpu.SemaphoreType.DMA],
    )(features_memory, labels_memory)
    return feats_out, labels_out
